# baseline (speedup 1.0000x reference)
import functools

import jax
import jax.numpy as jnp
from jax import lax
from jax.experimental import pallas as pl
from jax.experimental.pallas import tpu as pltpu
from jax.experimental.pallas import tpu_sc as plsc

N_ROWS = 8192
D = 4096
N_IDX = 4096
NC = 2
NS = 16
PER_SUB = N_IDX // (NC * NS)
BINS_PER_SUB = N_ROWS // NS
ROW_BLK = 1024


def _sc_hist(idx_hbm, s_hbm, w_hbm, den_hbm,
             idx_v, zidx_v, ssel_v, stage_v, shared, shared_d, sem):
    cid = lax.axis_index("c")
    sid = lax.axis_index("s")
    base = cid * (N_IDX // NC) + sid * PER_SUB

    def zero_chunk(k, _):
        stage_v[pl.ds(k * 16, 16)] = jnp.zeros((16,), jnp.float32)
        return 0
    lax.fori_loop(0, BINS_PER_SUB // 16, zero_chunk, 0)

    def zero_idx_chunk(k, _):
        zidx_v[pl.ds(k * 16, 16)] = jnp.zeros((16,), jnp.int32)
        return 0
    lax.fori_loop(0, PER_SUB // 16, zero_idx_chunk, 0)

    pltpu.sync_copy(stage_v, shared.at[pl.ds(sid * BINS_PER_SUB, BINS_PER_SUB)])
    pltpu.sync_copy(stage_v.at[pl.ds(0, 16)], shared_d)
    plsc.subcore_barrier()

    pltpu.sync_copy(idx_hbm.at[pl.ds(base, PER_SUB)], idx_v)
    pltpu.async_copy(s_hbm.at[idx_v], ssel_v, sem).wait()
    pltpu.sync_copy(ssel_v, shared.at[idx_v], add=True)
    pltpu.sync_copy(ssel_v, shared_d.at[zidx_v], add=True)
    plsc.subcore_barrier()

    pltpu.sync_copy(shared.at[pl.ds(sid * BINS_PER_SUB, BINS_PER_SUB)], stage_v)
    pltpu.sync_copy(stage_v, w_hbm.at[cid, pl.ds(sid * BINS_PER_SUB, BINS_PER_SUB)])

    @pl.when(sid == 0)
    def _pub_den():
        pltpu.sync_copy(shared_d, den_hbm.at[cid])


def _mv_body(w_ref, t_ref, den_ref, o_ref, acc_ref):
    pid = pl.program_id(0)

    @pl.when(pid == 0)
    def _init():
        acc_ref[...] = jnp.zeros_like(acc_ref)

    def slab(k, a):
        wsub = w_ref[0, pl.ds(k * 8, 8), :] + w_ref[1, pl.ds(k * 8, 8), :]
        return a + t_ref[pl.ds(k * 8, 8), :] * wsub

    part = lax.fori_loop(0, ROW_BLK // 8, slab,
                         jnp.zeros((8, D), jnp.float32))
    acc_ref[...] += part

    @pl.when(pid == pl.num_programs(0) - 1)
    def _fin():
        denom = jnp.sum(den_ref[...])
        o_ref[...] = (jnp.sum(acc_ref[...], axis=0, keepdims=True)
                      / (denom + 1e-6))


def kernel(tokens, s, idx):
    idx32 = idx.astype(jnp.int32)

    mesh = plsc.VectorSubcoreMesh(core_axis_name="c", subcore_axis_name="s")
    hist = functools.partial(
        pl.kernel,
        mesh=mesh,
        out_type=(
            jax.ShapeDtypeStruct((NC, N_ROWS), jnp.float32),
            jax.ShapeDtypeStruct((NC, 16), jnp.float32),
        ),
        scratch_types=[
            pltpu.VMEM((PER_SUB,), jnp.int32),
            pltpu.VMEM((PER_SUB,), jnp.int32),
            pltpu.VMEM((PER_SUB,), jnp.float32),
            pltpu.VMEM((BINS_PER_SUB,), jnp.float32),
            pltpu.VMEM_SHARED((N_ROWS,), jnp.float32),
            pltpu.VMEM_SHARED((16,), jnp.float32),
            pltpu.SemaphoreType.DMA,
        ],
    )(_sc_hist)
    w, den = hist(idx32, s)
    w3 = w.reshape(NC, N_ROWS, 1)

    out = pl.pallas_call(
        _mv_body,
        grid=(N_ROWS // ROW_BLK,),
        in_specs=[
            pl.BlockSpec((NC, ROW_BLK, 1), lambda i: (0, i, 0)),
            pl.BlockSpec((ROW_BLK, D), lambda i: (i, 0)),
            pl.BlockSpec((NC, 16), lambda i: (0, 0)),
        ],
        out_specs=pl.BlockSpec((1, D), lambda i: (0, 0)),
        out_shape=jax.ShapeDtypeStruct((1, D), jnp.float32),
        scratch_shapes=[pltpu.VMEM((8, D), jnp.float32)],
    )(w3, tokens, den)

    return out


# speedup vs baseline: 1.8155x; 1.8155x over previous
import functools

import jax
import jax.numpy as jnp
from jax import lax
from jax.experimental import pallas as pl
from jax.experimental.pallas import tpu as pltpu
from jax.experimental.pallas import tpu_sc as plsc

N_ROWS = 8192
D = 4096
N_IDX = 4096
NC = 2
NS = 16
PER_SUB = N_IDX // (NC * NS)
BINS_PER_SUB = N_ROWS // NS
ROW_BLK = 1024


def _sc_hist(idx_hbm, s_hbm, w_hbm, den_hbm,
             idx_v, zidx_v, ssel_v, stage_v, shared, shared_d, sem):
    cid = lax.axis_index("c")
    sid = lax.axis_index("s")
    base = cid * (N_IDX // NC) + sid * PER_SUB

    def zero_chunk(k, _):
        stage_v[pl.ds(k * 16, 16)] = jnp.zeros((16,), jnp.float32)
        return 0
    lax.fori_loop(0, BINS_PER_SUB // 16, zero_chunk, 0)

    def zero_idx_chunk(k, _):
        zidx_v[pl.ds(k * 16, 16)] = jnp.zeros((16,), jnp.int32)
        return 0
    lax.fori_loop(0, PER_SUB // 16, zero_idx_chunk, 0)

    pltpu.sync_copy(stage_v, shared.at[pl.ds(sid * BINS_PER_SUB, BINS_PER_SUB)])
    pltpu.sync_copy(stage_v.at[pl.ds(0, 16)], shared_d)
    plsc.subcore_barrier()

    pltpu.sync_copy(idx_hbm.at[pl.ds(base, PER_SUB)], idx_v)
    pltpu.async_copy(s_hbm.at[idx_v], ssel_v, sem).wait()
    pltpu.sync_copy(ssel_v, shared.at[idx_v], add=True)
    pltpu.sync_copy(ssel_v, shared_d.at[zidx_v], add=True)
    plsc.subcore_barrier()

    pltpu.sync_copy(shared.at[pl.ds(sid * BINS_PER_SUB, BINS_PER_SUB)], stage_v)
    pltpu.sync_copy(stage_v, w_hbm.at[cid, pl.ds(sid * BINS_PER_SUB, BINS_PER_SUB)])

    @pl.when(sid == 0)
    def _pub_den():
        pltpu.sync_copy(shared_d, den_hbm.at[cid])


def _mv_body(w_ref, t_ref, den_ref, o_ref, acc_ref):
    pid = pl.program_id(0)

    @pl.when(pid == 0)
    def _init():
        acc_ref[...] = jnp.zeros_like(acc_ref)

    wrow = w_ref[0, 0] + w_ref[1, 0]
    acc_ref[...] += jax.lax.dot_general(
        wrow, t_ref[...], (((1,), (0,)), ((), ())),
        preferred_element_type=jnp.float32)

    @pl.when(pid == pl.num_programs(0) - 1)
    def _fin():
        denom = jnp.sum(den_ref[...])
        o_ref[...] = acc_ref[...] / (denom + 1e-6)


def kernel(tokens, s, idx):
    idx32 = idx.astype(jnp.int32)

    mesh = plsc.VectorSubcoreMesh(core_axis_name="c", subcore_axis_name="s")
    hist = functools.partial(
        pl.kernel,
        mesh=mesh,
        out_type=(
            jax.ShapeDtypeStruct((NC, N_ROWS), jnp.float32),
            jax.ShapeDtypeStruct((NC, 16), jnp.float32),
        ),
        scratch_types=[
            pltpu.VMEM((PER_SUB,), jnp.int32),
            pltpu.VMEM((PER_SUB,), jnp.int32),
            pltpu.VMEM((PER_SUB,), jnp.float32),
            pltpu.VMEM((BINS_PER_SUB,), jnp.float32),
            pltpu.VMEM_SHARED((N_ROWS,), jnp.float32),
            pltpu.VMEM_SHARED((16,), jnp.float32),
            pltpu.SemaphoreType.DMA,
        ],
    )(_sc_hist)
    w, den = hist(idx32, s)
    w4 = w.reshape(NC, N_ROWS // ROW_BLK, 1, ROW_BLK)

    out = pl.pallas_call(
        _mv_body,
        grid=(N_ROWS // ROW_BLK,),
        in_specs=[
            pl.BlockSpec((NC, 1, 1, ROW_BLK), lambda i: (0, i, 0, 0)),
            pl.BlockSpec((ROW_BLK, D), lambda i: (i, 0)),
            pl.BlockSpec((NC, 16), lambda i: (0, 0)),
        ],
        out_specs=pl.BlockSpec((1, D), lambda i: (0, 0)),
        out_shape=jax.ShapeDtypeStruct((1, D), jnp.float32),
        scratch_shapes=[pltpu.VMEM((1, D), jnp.float32)],
    )(w4, tokens, den)

    return out


# speedup vs baseline: 1.8531x; 1.0207x over previous
import functools

import jax
import jax.numpy as jnp
from jax import lax
from jax.experimental import pallas as pl
from jax.experimental.pallas import tpu as pltpu
from jax.experimental.pallas import tpu_sc as plsc

N_ROWS = 8192
D = 4096
N_IDX = 4096
NC = 2
NS = 16
PER_SUB = N_IDX // (NC * NS)
BINS_PER_SUB = N_ROWS // NS
ROW_BLK = 512


def _sc_hist(idx_hbm, s_hbm, w_hbm, den_hbm,
             idx_v, zidx_v, ssel_v, stage_v, shared, shared_d, sem):
    cid = lax.axis_index("c")
    sid = lax.axis_index("s")
    base = cid * (N_IDX // NC) + sid * PER_SUB

    def zero_chunk(k, _):
        stage_v[pl.ds(k * 16, 16)] = jnp.zeros((16,), jnp.float32)
        return 0
    lax.fori_loop(0, BINS_PER_SUB // 16, zero_chunk, 0)

    def zero_idx_chunk(k, _):
        zidx_v[pl.ds(k * 16, 16)] = jnp.zeros((16,), jnp.int32)
        return 0
    lax.fori_loop(0, PER_SUB // 16, zero_idx_chunk, 0)

    pltpu.sync_copy(stage_v, shared.at[pl.ds(sid * BINS_PER_SUB, BINS_PER_SUB)])
    pltpu.sync_copy(stage_v.at[pl.ds(0, 16)], shared_d)
    plsc.subcore_barrier()

    pltpu.sync_copy(idx_hbm.at[pl.ds(base, PER_SUB)], idx_v)
    pltpu.async_copy(s_hbm.at[idx_v], ssel_v, sem).wait()
    pltpu.sync_copy(ssel_v, shared.at[idx_v], add=True)
    pltpu.sync_copy(ssel_v, shared_d.at[zidx_v], add=True)
    plsc.subcore_barrier()

    pltpu.sync_copy(shared.at[pl.ds(sid * BINS_PER_SUB, BINS_PER_SUB)], stage_v)
    pltpu.sync_copy(stage_v, w_hbm.at[cid, pl.ds(sid * BINS_PER_SUB, BINS_PER_SUB)])

    @pl.when(sid == 0)
    def _pub_den():
        pltpu.sync_copy(shared_d, den_hbm.at[cid])


def _mv_body(w_ref, t_ref, den_ref, o_ref, acc_ref):
    pid = pl.program_id(0)

    @pl.when(pid == 0)
    def _init():
        acc_ref[...] = jnp.zeros_like(acc_ref)

    wrow = w_ref[0, 0] + w_ref[1, 0]
    acc_ref[...] += jax.lax.dot_general(
        wrow, t_ref[...], (((1,), (0,)), ((), ())),
        preferred_element_type=jnp.float32)

    @pl.when(pid == pl.num_programs(0) - 1)
    def _fin():
        denom = jnp.sum(den_ref[...])
        o_ref[...] = acc_ref[...] / (denom + 1e-6)


def kernel(tokens, s, idx):
    idx32 = idx.astype(jnp.int32)

    mesh = plsc.VectorSubcoreMesh(core_axis_name="c", subcore_axis_name="s")
    hist = functools.partial(
        pl.kernel,
        mesh=mesh,
        out_type=(
            jax.ShapeDtypeStruct((NC, N_ROWS), jnp.float32),
            jax.ShapeDtypeStruct((NC, 16), jnp.float32),
        ),
        scratch_types=[
            pltpu.VMEM((PER_SUB,), jnp.int32),
            pltpu.VMEM((PER_SUB,), jnp.int32),
            pltpu.VMEM((PER_SUB,), jnp.float32),
            pltpu.VMEM((BINS_PER_SUB,), jnp.float32),
            pltpu.VMEM_SHARED((N_ROWS,), jnp.float32),
            pltpu.VMEM_SHARED((16,), jnp.float32),
            pltpu.SemaphoreType.DMA,
        ],
    )(_sc_hist)
    w, den = hist(idx32, s)
    w4 = w.reshape(NC, N_ROWS // ROW_BLK, 1, ROW_BLK)

    out = pl.pallas_call(
        _mv_body,
        grid=(N_ROWS // ROW_BLK,),
        in_specs=[
            pl.BlockSpec((NC, 1, 1, ROW_BLK), lambda i: (0, i, 0, 0)),
            pl.BlockSpec((ROW_BLK, D), lambda i: (i, 0)),
            pl.BlockSpec((NC, 16), lambda i: (0, 0)),
        ],
        out_specs=pl.BlockSpec((1, D), lambda i: (0, 0)),
        out_shape=jax.ShapeDtypeStruct((1, D), jnp.float32),
        scratch_shapes=[pltpu.VMEM((1, D), jnp.float32)],
    )(w4, tokens, den)

    return out
